# Initial kernel scaffold; baseline (speedup 1.0000x reference)
import jax
import jax.numpy as jnp
from jax.experimental import pallas as pl


def kernel(input_ids, word_table, pos_table, type_table, gamma, beta):
    raise NotImplementedError("write your pallas kernel here")


# speedup vs baseline: 14.5742x; 14.5742x over previous
import functools

import jax
import jax.numpy as jnp
from jax import lax
from jax.experimental import pallas as pl
from jax.experimental.pallas import tpu as pltpu
from jax.experimental.pallas import tpu_sc as plsc

D = 128
L = 200
VOCAB = 178
VPAD = 184
B = 1024
LBLK = 8
NW = 32
PER_W = (B * L) // NW
NROW = PER_W // 128


def _ctable_body(word_ref, pos_ref, type_ref, gamma_ref, beta_ref, out_ref):
    x = (word_ref[...][None, :, :]
         + pos_ref[...][:, None, :]
         + type_ref[...][0:1][None, :, :])
    mean = jnp.mean(x, axis=-1, keepdims=True)
    var = jnp.mean(jnp.square(x - mean), axis=-1, keepdims=True)
    y = (x - mean) * lax.rsqrt(var + 1e-5)
    out_ref[...] = (y * gamma_ref[...][None, :, :]
                    + beta_ref[...][None, :, :])


def _make_ctable(word_pad, pos_used, type_table, gamma2d, beta2d):
    return pl.pallas_call(
        _ctable_body,
        grid=(L // LBLK,),
        in_specs=[
            pl.BlockSpec((VPAD, D), lambda i: (0, 0)),
            pl.BlockSpec((LBLK, D), lambda i: (i, 0)),
            pl.BlockSpec((2, D), lambda i: (0, 0)),
            pl.BlockSpec((1, D), lambda i: (0, 0)),
            pl.BlockSpec((1, D), lambda i: (0, 0)),
        ],
        out_specs=pl.BlockSpec((LBLK, VPAD, D), lambda i: (i, 0, 0)),
        out_shape=jax.ShapeDtypeStruct((L, VPAD, D), jnp.float32),
    )(word_pad, pos_used, type_table, gamma2d, beta2d)


def _sc_gather_body(ctable_hbm, ids_hbm, out_hbm, ids_v, idx_v, buf_v, sem):
    wid = lax.axis_index("s") * 2 + lax.axis_index("c")
    base = wid * PER_W
    pltpu.sync_copy(ids_hbm.at[pl.ds(base, PER_W)], ids_v)

    def idx_body(row, carry):
        for sub in range(8):
            off = row * 128 + sub * 16
            p = base + off + lax.iota(jnp.int32, 16)
            lpos = lax.rem(p, L)
            idx_v[row, pl.ds(sub * 16, 16)] = ids_v[pl.ds(off, 16)] + lpos * VPAD
        return carry

    lax.fori_loop(0, NROW, idx_body, 0)

    def mv_body(row, carry):
        pltpu.async_copy(ctable_hbm.at[idx_v.at[row]], buf_v, sem).wait()
        pltpu.sync_copy(buf_v, out_hbm.at[pl.ds(base + row * 128, 128)])
        return carry

    lax.fori_loop(0, NROW, mv_body, 0)


@functools.cache
def _sc_gather():
    return pl.kernel(
        _sc_gather_body,
        out_type=jax.ShapeDtypeStruct((B * L, D), jnp.float32),
        mesh=plsc.VectorSubcoreMesh(core_axis_name="c", subcore_axis_name="s"),
        scratch_types=[
            pltpu.VMEM((PER_W,), jnp.int32),
            pltpu.VMEM((NROW, 128), jnp.int32),
            pltpu.VMEM((128, D), jnp.float32),
            pltpu.SemaphoreType.DMA,
        ],
    )


def kernel(input_ids, word_table, pos_table, type_table, gamma, beta):
    ids_flat = input_ids.reshape(-1).astype(jnp.int32)
    word_pad = jnp.pad(word_table, ((0, VPAD - VOCAB), (0, 0)))
    ctable = _make_ctable(word_pad, pos_table[:L], type_table,
                          gamma.reshape(1, D), beta.reshape(1, D))
    out_flat = _sc_gather()(ctable.reshape(L * VPAD, D), ids_flat)
    return out_flat.reshape(B, L, D)


# speedup vs baseline: 18.7280x; 1.2850x over previous
import functools

import jax
import jax.numpy as jnp
from jax import lax
from jax.experimental import pallas as pl
from jax.experimental.pallas import tpu as pltpu
from jax.experimental.pallas import tpu_sc as plsc

D = 128
L = 200
VOCAB = 178
VPAD = 184
B = 1024
LBLK = 8
NW = 32
PER_W = (B * L) // NW
NROW = PER_W // 128


def _ctable_body(word_ref, pos_ref, type_ref, gamma_ref, beta_ref, out_ref):
    x = (word_ref[...][None, :, :]
         + pos_ref[...][:, None, :]
         + type_ref[...][0:1][None, :, :])
    mean = jnp.mean(x, axis=-1, keepdims=True)
    var = jnp.mean(jnp.square(x - mean), axis=-1, keepdims=True)
    y = (x - mean) * lax.rsqrt(var + 1e-5)
    out_ref[...] = (y * gamma_ref[...][None, :, :]
                    + beta_ref[...][None, :, :])


def _make_ctable(word_pad, pos_used, type_table, gamma2d, beta2d):
    return pl.pallas_call(
        _ctable_body,
        grid=(L // LBLK,),
        in_specs=[
            pl.BlockSpec((VPAD, D), lambda i: (0, 0)),
            pl.BlockSpec((LBLK, D), lambda i: (i, 0)),
            pl.BlockSpec((2, D), lambda i: (0, 0)),
            pl.BlockSpec((1, D), lambda i: (0, 0)),
            pl.BlockSpec((1, D), lambda i: (0, 0)),
        ],
        out_specs=pl.BlockSpec((LBLK, VPAD, D), lambda i: (i, 0, 0)),
        out_shape=jax.ShapeDtypeStruct((L, VPAD, D), jnp.float32),
    )(word_pad, pos_used, type_table, gamma2d, beta2d)


NBUF = 5
GROUPS = NROW // NBUF


def _sc_gather_body(ctable_hbm, ids_hbm, out_hbm, ids_v, idx_v, bufs_v,
                    gsem, osem):
    wid = lax.axis_index("s") * 2 + lax.axis_index("c")
    base = wid * PER_W
    pltpu.sync_copy(ids_hbm.at[pl.ds(base, PER_W)], ids_v)

    def idx_body(row, carry):
        for sub in range(8):
            off = row * 128 + sub * 16
            p = base + off + lax.iota(jnp.int32, 16)
            lpos = lax.rem(p, L)
            idx_v[row, pl.ds(sub * 16, 16)] = ids_v[pl.ds(off, 16)] + lpos * VPAD
        return carry

    lax.fori_loop(0, NROW, idx_body, 0)

    def gather_desc(row, b):
        return pltpu.make_async_copy(
            ctable_hbm.at[idx_v.at[row]], bufs_v.at[b], gsem.at[b])

    def write_desc(row, b):
        return pltpu.make_async_copy(
            bufs_v.at[b], out_hbm.at[pl.ds(base + row * 128, 128)], osem.at[b])

    def grp_body(g, carry):
        for b in range(NBUF):
            row = g * NBUF + b

            @pl.when(g > 0)
            def _():
                write_desc(row - NBUF, b).wait()

            gather_desc(row, b).start()
        for b in range(NBUF):
            row = g * NBUF + b
            gather_desc(row, b).wait()
            write_desc(row, b).start()
        return carry

    lax.fori_loop(0, GROUPS, grp_body, 0)
    for b in range(NBUF):
        write_desc((GROUPS - 1) * NBUF + b, b).wait()


@functools.cache
def _sc_gather():
    return pl.kernel(
        _sc_gather_body,
        out_type=jax.ShapeDtypeStruct((B * L, D), jnp.float32),
        mesh=plsc.VectorSubcoreMesh(core_axis_name="c", subcore_axis_name="s"),
        scratch_types=[
            pltpu.VMEM((PER_W,), jnp.int32),
            pltpu.VMEM((NROW, 128), jnp.int32),
            pltpu.VMEM((NBUF, 128, D), jnp.float32),
            pltpu.SemaphoreType.DMA((NBUF,)),
            pltpu.SemaphoreType.DMA((NBUF,)),
        ],
    )


def kernel(input_ids, word_table, pos_table, type_table, gamma, beta):
    ids_flat = input_ids.reshape(-1).astype(jnp.int32)
    word_pad = jnp.pad(word_table, ((0, VPAD - VOCAB), (0, 0)))
    ctable = _make_ctable(word_pad, pos_table[:L], type_table,
                          gamma.reshape(1, D), beta.reshape(1, D))
    out_flat = _sc_gather()(ctable.reshape(L * VPAD, D), ids_flat)
    return out_flat.reshape(B, L, D)


# speedup vs baseline: 20.0346x; 1.0698x over previous
import functools

import jax
import jax.numpy as jnp
from jax import lax
from jax.experimental import pallas as pl
from jax.experimental.pallas import tpu as pltpu
from jax.experimental.pallas import tpu_sc as plsc

D = 128
L = 200
VOCAB = 178
VPAD = 184
B = 1024
LBLK = 40
NW = 32
PER_W = (B * L) // NW
NROW = PER_W // 128


def _ctable_body(word_ref, pos_ref, type_ref, gamma_ref, beta_ref, out_ref):
    w = word_ref[...] + type_ref[...][0:1]
    p = pos_ref[...]
    sw = jnp.sum(w, axis=1)
    sp = jnp.sum(p, axis=1)
    sw2 = jnp.sum(w * w, axis=1)
    sp2 = jnp.sum(p * p, axis=1)
    cross = lax.dot_general(p, w, (((1,), (1,)), ((), ())),
                            precision=lax.Precision.HIGHEST)
    s1 = sp[:, None] + sw[None, :]
    s2 = sp2[:, None] + sw2[None, :] + 2.0 * cross
    mean = s1 * (1.0 / D)
    var = s2 * (1.0 / D) - mean * mean
    scale = lax.rsqrt(var + 1e-5)
    x = p[:, None, :] + w[None, :, :]
    y = (x - mean[:, :, None]) * scale[:, :, None]
    out_ref[...] = (y * gamma_ref[...][None, :, :]
                    + beta_ref[...][None, :, :])


def _make_ctable(word_pad, pos_table, type_table, gamma2d, beta2d):
    return pl.pallas_call(
        _ctable_body,
        grid=(L // LBLK,),
        in_specs=[
            pl.BlockSpec((VPAD, D), lambda i: (0, 0)),
            pl.BlockSpec((LBLK, D), lambda i: (i, 0)),
            pl.BlockSpec((2, D), lambda i: (0, 0)),
            pl.BlockSpec((1, D), lambda i: (0, 0)),
            pl.BlockSpec((1, D), lambda i: (0, 0)),
        ],
        out_specs=pl.BlockSpec((LBLK, VPAD, D), lambda i: (i, 0, 0)),
        out_shape=jax.ShapeDtypeStruct((L, VPAD, D), jnp.float32),
    )(word_pad, pos_table, type_table, gamma2d, beta2d)


NBUF = 5
GROUPS = NROW // NBUF


def _sc_gather_body(ctable_hbm, ids_hbm, out_hbm, ids_v, idx_v, bufs_v,
                    gsem, osem):
    wid = lax.axis_index("s") * 2 + lax.axis_index("c")
    base = wid * PER_W
    pltpu.sync_copy(ids_hbm.at[pl.ds(base, PER_W)], ids_v)

    def idx_body(row, carry):
        for sub in range(8):
            off = row * 128 + sub * 16
            p = base + off + lax.iota(jnp.int32, 16)
            lpos = lax.rem(p, L)
            idx_v[row, pl.ds(sub * 16, 16)] = ids_v[pl.ds(off, 16)] + lpos * VPAD
        return carry

    lax.fori_loop(0, NROW, idx_body, 0)

    def gather_desc(row, b):
        return pltpu.make_async_copy(
            ctable_hbm.at[idx_v.at[row]], bufs_v.at[b], gsem.at[b])

    def write_desc(row, b):
        return pltpu.make_async_copy(
            bufs_v.at[b], out_hbm.at[pl.ds(base + row * 128, 128)], osem.at[b])

    def grp_body(g, carry):
        for b in range(NBUF):
            row = g * NBUF + b

            @pl.when(g > 0)
            def _():
                write_desc(row - NBUF, b).wait()

            gather_desc(row, b).start()
        for b in range(NBUF):
            row = g * NBUF + b
            gather_desc(row, b).wait()
            write_desc(row, b).start()
        return carry

    lax.fori_loop(0, GROUPS, grp_body, 0)
    for b in range(NBUF):
        write_desc((GROUPS - 1) * NBUF + b, b).wait()


@functools.cache
def _sc_gather():
    return pl.kernel(
        _sc_gather_body,
        out_type=jax.ShapeDtypeStruct((B * L, D), jnp.float32),
        mesh=plsc.VectorSubcoreMesh(core_axis_name="c", subcore_axis_name="s"),
        scratch_types=[
            pltpu.VMEM((PER_W,), jnp.int32),
            pltpu.VMEM((NROW, 128), jnp.int32),
            pltpu.VMEM((NBUF, 128, D), jnp.float32),
            pltpu.SemaphoreType.DMA((NBUF,)),
            pltpu.SemaphoreType.DMA((NBUF,)),
        ],
    )


def kernel(input_ids, word_table, pos_table, type_table, gamma, beta):
    ids_flat = input_ids.reshape(-1).astype(jnp.int32)
    word_pad = jnp.pad(word_table, ((0, VPAD - VOCAB), (0, 0)))
    ctable = _make_ctable(word_pad, pos_table, type_table,
                          gamma.reshape(1, D), beta.reshape(1, D))
    out_flat = _sc_gather()(ctable.reshape(L * VPAD, D), ids_flat)
    return out_flat.reshape(B, L, D)
